# Initial kernel scaffold; baseline (speedup 1.0000x reference)
import jax
import jax.numpy as jnp
from jax.experimental import pallas as pl


def kernel(col, row, weight, port, col_embed, row_embed, W_weight, b_weight, W_port, b_port):
    raise NotImplementedError("write your pallas kernel here")


# speedup vs baseline: 2.6596x; 2.6596x over previous
import jax
import jax.numpy as jnp
from jax.experimental import pallas as pl

_B = 16384
_D = 32
_BLK = 2048


def _body(col_ref, row_ref, w_ref, port_ref, m16_ref, mp_ref, wv_ref, bias_ref, out_ref):
    c = col_ref[:]
    r = row_ref[:]
    io = jax.lax.broadcasted_iota(jnp.int32, (_BLK, 16), 1)
    oh = jnp.where((c == io) | ((r + 8) == io), 1.0, 0.0)
    acc = jnp.dot(oh, m16_ref[:], preferred_element_type=jnp.float32)
    acc = acc + jnp.dot(port_ref[:], mp_ref[:], preferred_element_type=jnp.float32)
    acc = acc + w_ref[:] * wv_ref[:]
    out_ref[:] = acc + bias_ref[:]


def kernel(col, row, weight, port, col_embed, row_embed, W_weight, b_weight, W_port, b_port):
    col2 = col.reshape(_B, 1).astype(jnp.int32)
    row2 = row.reshape(_B, 1).astype(jnp.int32)
    m16 = jnp.zeros((16, _D), jnp.float32)
    m16 = m16.at[0:6].set(col_embed).at[8:14].set(row_embed)
    mp = W_port.T
    wv = W_weight.T
    bias = (b_weight + b_port).reshape(1, _D)

    grid = _B // _BLK
    return pl.pallas_call(
        _body,
        grid=(grid,),
        in_specs=[
            pl.BlockSpec((_BLK, 1), lambda i: (i, 0)),
            pl.BlockSpec((_BLK, 1), lambda i: (i, 0)),
            pl.BlockSpec((_BLK, 1), lambda i: (i, 0)),
            pl.BlockSpec((_BLK, 6), lambda i: (i, 0)),
            pl.BlockSpec((16, _D), lambda i: (0, 0)),
            pl.BlockSpec((6, _D), lambda i: (0, 0)),
            pl.BlockSpec((1, _D), lambda i: (0, 0)),
            pl.BlockSpec((1, _D), lambda i: (0, 0)),
        ],
        out_specs=pl.BlockSpec((_BLK, _D), lambda i: (i, 0)),
        out_shape=jax.ShapeDtypeStruct((_B, _D), jnp.float32),
    )(col2, row2, weight, port, m16, mp, wv, bias)
